# baseline (speedup 1.0000x reference)
import dataclasses
import functools

import jax
import jax.numpy as jnp
from jax import lax
from jax.experimental import pallas as pl
from jax.experimental.pallas import tpu as pltpu
from jax.experimental.pallas import tpu_sc as plsc

N = 2048
G = 32
C0 = 64
F = 32
E = 65536

NS = 16
NW = 2 * NS
ROWS_W = N // NW
HALF_ROWS = ROWS_W // 2
HALF_W = HALF_ROWS * N
EPIECE = 16384
NPIECE = E // EPIECE


def _build_c(eidx):
    cp = pltpu.CompilerParams()
    if "needs_layout_passes" in pltpu.CompilerParams.__dataclass_fields__:
        cp = dataclasses.replace(cp, needs_layout_passes=False)

    @functools.partial(
        pl.kernel,
        out_type=jax.ShapeDtypeStruct((N * N,), jnp.float32),
        mesh=plsc.VectorSubcoreMesh(core_axis_name="c", subcore_axis_name="s"),
        compiler_params=cp,
        scratch_types=[
            pltpu.VMEM((HALF_W,), jnp.float32),
            pltpu.VMEM((EPIECE,), jnp.int32),
            pltpu.VMEM((EPIECE,), jnp.int32),
            pltpu.SemaphoreType.DMA,
            pltpu.SemaphoreType.DMA,
            pltpu.SemaphoreType.DMA,
        ],
    )
    def k(eidx_hbm, c_hbm, cbuf, eb0, eb1, sem0, sem1, semo):
        wid = lax.axis_index("c") * NS + lax.axis_index("s")
        zero16 = jnp.zeros((16,), jnp.float32)
        one16 = jnp.ones((16,), jnp.float32)
        bufs = (eb0, eb1)
        sems = (sem0, sem1)

        @pl.loop(0, 2)
        def _(half):
            basew = (wid * ROWS_W + half * HALF_ROWS) * N

            @pl.loop(0, HALF_W, step=64)
            def _(off):
                for j in range(0, 64, 16):
                    cbuf[pl.ds(off + j, 16)] = zero16

            cps = [None] * NPIECE
            cps[0] = pltpu.async_copy(
                eidx_hbm.at[pl.ds(0, EPIECE)], eb0, sem0)
            for i in range(NPIECE):
                if i + 1 < NPIECE:
                    cps[i + 1] = pltpu.async_copy(
                        eidx_hbm.at[pl.ds((i + 1) * EPIECE, EPIECE)],
                        bufs[(i + 1) % 2], sems[(i + 1) % 2])
                cps[i].wait()
                buf = bufs[i % 2]

                @pl.loop(0, EPIECE, step=64)
                def _(v, buf=buf):
                    for j in range(0, 64, 16):
                        r = buf[pl.ds(v + j, 16)] - basew
                        mask = (r >= 0) & (r < HALF_W)
                        plsc.addupdate_scatter(cbuf, [r], one16, mask=mask)

            pltpu.async_copy(
                cbuf, c_hbm.at[pl.ds(basew, HALF_W)], semo
            ).wait()

    return k(eidx)


def _mm_body(ci, x_ref, w_ref, o_ref):
    w = w_ref[...]
    for g in range(G):
        o_ref[:, g * F:(g + 1) * F] = jnp.dot(
            x_ref[:, g * ci:(g + 1) * ci], w,
            preferred_element_type=jnp.float32)


def _mm(x, w):
    ci = w.shape[0]
    return pl.pallas_call(
        functools.partial(_mm_body, ci),
        out_shape=jax.ShapeDtypeStruct((N, G * F), jnp.float32),
    )(x, w)


def _agg_body(c_ref, hw_ref, gam_ref, bet_ref, t_ref, tt_ref, o_ref):
    c = c_ref[...]
    hw = hw_ref[...]
    deg = jnp.sum(c, axis=1, keepdims=True) + 2.0
    dis = jax.lax.rsqrt(deg)
    out = jnp.dot(c, dis * hw, preferred_element_type=jnp.float32)
    out = dis * out + (2.0 / deg) * hw
    n_tot = jnp.float32(G * N)
    t = t_ref[...]
    s = jnp.sum(out, axis=0, keepdims=True)
    ss = jnp.sum(out * out, axis=0, keepdims=True)
    s_f = jnp.dot(s, t, preferred_element_type=jnp.float32)
    ss_f = jnp.dot(ss, t, preferred_element_type=jnp.float32)
    mu = s_f / n_tot
    var = ss_f / n_tot - mu * mu
    scale = gam_ref[...] * jax.lax.rsqrt(var + 1e-5)
    shift = bet_ref[...] - mu * scale
    colscale = jnp.dot(scale, tt_ref[...], preferred_element_type=jnp.float32)
    colshift = jnp.dot(shift, tt_ref[...], preferred_element_type=jnp.float32)
    o_ref[...] = jnp.maximum(out * colscale + colshift, 0.0)


def _agg(c, hw, gam, bet, t, tt):
    return pl.pallas_call(
        _agg_body,
        out_shape=jax.ShapeDtypeStruct((N, G * F), jnp.float32),
    )(c, hw, gam, bet, t, tt)


def kernel(x, edge_index, W1, b1, g1, be1, W2, b2, g2, be2, W3, b3, g3, be3):
    eidx = edge_index[1] * N + edge_index[0]
    c = _build_c(eidx).reshape(N, N)
    h0 = x.reshape(G * C0, N).T
    t = jnp.tile(jnp.eye(F, dtype=jnp.float32), (G, 1))
    tt = t.T
    H = h0
    for w, gam, bet in ((W1, g1, be1), (W2, g2, be2), (W3, g3, be3)):
        hw = _mm(H, w)
        H = _agg(c, hw, gam.reshape(1, F), bet.reshape(1, F), t, tt)
    return H.reshape(N, G, F).transpose(1, 2, 0)


# speedup vs baseline: 1.3633x; 1.3633x over previous
import dataclasses
import functools

import jax
import jax.numpy as jnp
from jax import lax
from jax.experimental import pallas as pl
from jax.experimental.pallas import tpu as pltpu
from jax.experimental.pallas import tpu_sc as plsc

N = 2048
G = 32
C0 = 64
F = 32
E = 65536

NS = 16
NW = 2 * NS
ROWS_W = N // NW
HALF_ROWS = ROWS_W // 2
HALF_W = HALF_ROWS * N
EPIECE = 16384
NPIECE = E // EPIECE


def _build_c(eidx):
    cp = pltpu.CompilerParams()
    if "needs_layout_passes" in pltpu.CompilerParams.__dataclass_fields__:
        cp = dataclasses.replace(cp, needs_layout_passes=False)

    @functools.partial(
        pl.kernel,
        out_type=jax.ShapeDtypeStruct((N * N,), jnp.float32),
        mesh=plsc.VectorSubcoreMesh(core_axis_name="c", subcore_axis_name="s"),
        compiler_params=cp,
        scratch_types=[
            pltpu.VMEM((HALF_W,), jnp.float32),
            pltpu.VMEM((EPIECE,), jnp.int32),
            pltpu.VMEM((EPIECE,), jnp.int32),
            pltpu.SemaphoreType.DMA,
            pltpu.SemaphoreType.DMA,
            pltpu.SemaphoreType.DMA,
        ],
    )
    def k(eidx_hbm, c_hbm, cbuf, eb0, eb1, sem0, sem1, semo):
        wid = lax.axis_index("c") * NS + lax.axis_index("s")
        zero16 = jnp.zeros((16,), jnp.float32)
        one16 = jnp.ones((16,), jnp.float32)
        bufs = (eb0, eb1)
        sems = (sem0, sem1)

        @pl.loop(0, 2)
        def _(half):
            basew = (wid * ROWS_W + half * HALF_ROWS) * N

            @pl.loop(0, HALF_W, step=64)
            def _(off):
                for j in range(0, 64, 16):
                    cbuf[pl.ds(off + j, 16)] = zero16

            cps = [None] * NPIECE
            cps[0] = pltpu.async_copy(
                eidx_hbm.at[pl.ds(0, EPIECE)], eb0, sem0)
            for i in range(NPIECE):
                if i + 1 < NPIECE:
                    cps[i + 1] = pltpu.async_copy(
                        eidx_hbm.at[pl.ds((i + 1) * EPIECE, EPIECE)],
                        bufs[(i + 1) % 2], sems[(i + 1) % 2])
                cps[i].wait()
                buf = bufs[i % 2]

                @pl.loop(0, EPIECE, step=64)
                def _(v, buf=buf):
                    for j in range(0, 64, 16):
                        r = buf[pl.ds(v + j, 16)] - basew
                        mask = (r >= 0) & (r < HALF_W)
                        plsc.addupdate_scatter(cbuf, [r], one16, mask=mask)

            pltpu.async_copy(
                cbuf, c_hbm.at[pl.ds(basew, HALF_W)], semo
            ).wait()

    return k(eidx)


def _layer_body(ci, x_ref, wt_ref, gam_ref, bet_ref, ct_ref, t_ref, tt_ref,
                o_ref):
    ct = ct_ref[...]
    deg = jnp.sum(ct, axis=0, keepdims=True) + 2.0
    dis = jax.lax.rsqrt(deg)
    wt = wt_ref[...]
    hw = jnp.concatenate(
        [jnp.dot(wt, x_ref[g * ci:(g + 1) * ci, :],
                 preferred_element_type=jnp.float32) for g in range(G)],
        axis=0)
    out = jnp.dot(dis * hw, ct, preferred_element_type=jnp.float32)
    out = dis * out + (2.0 / deg) * hw
    n_tot = jnp.float32(G * N)
    rs = jnp.sum(out, axis=1, keepdims=True)
    rss = jnp.sum(out * out, axis=1, keepdims=True)
    s_f = jnp.dot(tt_ref[...], rs, preferred_element_type=jnp.float32)
    ss_f = jnp.dot(tt_ref[...], rss, preferred_element_type=jnp.float32)
    mu = s_f / n_tot
    var = ss_f / n_tot - mu * mu
    scale = gam_ref[...] * jax.lax.rsqrt(var + 1e-5)
    shift = bet_ref[...] - mu * scale
    rowscale = jnp.dot(t_ref[...], scale, preferred_element_type=jnp.float32)
    rowshift = jnp.dot(t_ref[...], shift, preferred_element_type=jnp.float32)
    o_ref[...] = jnp.maximum(out * rowscale + rowshift, 0.0)


def _layer(x, w, gam, bet, ct, t, tt):
    ci = w.shape[0]
    return pl.pallas_call(
        functools.partial(_layer_body, ci),
        out_shape=jax.ShapeDtypeStruct((G * F, N), jnp.float32),
    )(x, w.T, gam.reshape(F, 1), bet.reshape(F, 1), ct, t, tt)


def kernel(x, edge_index, W1, b1, g1, be1, W2, b2, g2, be2, W3, b3, g3, be3):
    eidx = edge_index[0] * N + edge_index[1]
    ct = _build_c(eidx).reshape(N, N)
    t = jnp.tile(jnp.eye(F, dtype=jnp.float32), (G, 1))
    tt = t.T
    H = x.reshape(G * C0, N)
    for w, gam, bet in ((W1, g1, be1), (W2, g2, be2), (W3, g3, be3)):
        H = _layer(H, w, gam, bet, ct, t, tt)
    return H.reshape(G, F, N)


# speedup vs baseline: 1.5570x; 1.1421x over previous
import dataclasses
import functools

import jax
import jax.numpy as jnp
from jax import lax
from jax.experimental import pallas as pl
from jax.experimental.pallas import tpu as pltpu
from jax.experimental.pallas import tpu_sc as plsc

N = 2048
G = 32
C0 = 64
F = 32
E = 65536

NS = 16
NW = 2 * NS
ROWS_W = N // NW
HALF_ROWS = ROWS_W // 2
HALF_W = HALF_ROWS * N
EPIECE = 16384
NPIECE = E // EPIECE


def _build_c(eidx):
    cp = pltpu.CompilerParams()
    if "needs_layout_passes" in pltpu.CompilerParams.__dataclass_fields__:
        cp = dataclasses.replace(cp, needs_layout_passes=False)

    @functools.partial(
        pl.kernel,
        out_type=jax.ShapeDtypeStruct((N * N,), jnp.float32),
        mesh=plsc.VectorSubcoreMesh(core_axis_name="c", subcore_axis_name="s"),
        compiler_params=cp,
        scratch_types=[
            pltpu.VMEM((HALF_W,), jnp.float32),
            pltpu.VMEM((EPIECE,), jnp.int32),
            pltpu.VMEM((EPIECE,), jnp.int32),
            pltpu.SemaphoreType.DMA,
            pltpu.SemaphoreType.DMA,
            pltpu.SemaphoreType.DMA,
        ],
    )
    def k(eidx_hbm, c_hbm, cbuf, eb0, eb1, sem0, sem1, semo):
        wid = lax.axis_index("c") * NS + lax.axis_index("s")
        zero16 = jnp.zeros((16,), jnp.float32)
        one16 = jnp.ones((16,), jnp.float32)
        bufs = (eb0, eb1)
        sems = (sem0, sem1)

        @pl.loop(0, 2)
        def _(half):
            basew = (wid * ROWS_W + half * HALF_ROWS) * N

            @pl.loop(0, HALF_W, step=64)
            def _(off):
                for j in range(0, 64, 16):
                    cbuf[pl.ds(off + j, 16)] = zero16

            cps = [None] * NPIECE
            cps[0] = pltpu.async_copy(
                eidx_hbm.at[pl.ds(0, EPIECE)], eb0, sem0)
            for i in range(NPIECE):
                if i + 1 < NPIECE:
                    cps[i + 1] = pltpu.async_copy(
                        eidx_hbm.at[pl.ds((i + 1) * EPIECE, EPIECE)],
                        bufs[(i + 1) % 2], sems[(i + 1) % 2])
                cps[i].wait()
                buf = bufs[i % 2]

                @pl.loop(0, EPIECE, step=64)
                def _(v, buf=buf):
                    for j in range(0, 64, 16):
                        r = buf[pl.ds(v + j, 16)] - basew
                        mask = (r >= 0) & (r < HALF_W)
                        plsc.addupdate_scatter(cbuf, [r], one16, mask=mask)

            pltpu.async_copy(
                cbuf, c_hbm.at[pl.ds(basew, HALF_W)], semo
            ).wait()

    return k(eidx)


def _transform(h, wt, ci):
    return jnp.concatenate(
        [jnp.dot(wt, h[g * ci:(g + 1) * ci, :],
                 preferred_element_type=jnp.float32) for g in range(G)],
        axis=0)


def _agg_bn(hw, ct, deg, dis, gam, bet):
    out = jnp.dot(dis * hw, ct, preferred_element_type=jnp.float32)
    out = dis * out + (2.0 / deg) * hw
    o3 = out.reshape(G, F, N)
    mu = jnp.mean(o3, axis=(0, 2))
    var = jnp.mean(o3 * o3, axis=(0, 2)) - mu * mu
    scale = gam * jax.lax.rsqrt(var + 1e-5)
    shift = bet - mu * scale
    o3 = jnp.maximum(o3 * scale[None, :, None] + shift[None, :, None], 0.0)
    return o3.reshape(G * F, N)


def _t1_body(x_ref, wt_ref, o_ref):
    o_ref[...] = _transform(x_ref[...], wt_ref[...], C0)


def _fused_body(hw1_ref, ct_ref, wt2_ref, wt3_ref, g1_ref, be1_ref,
                g2_ref, be2_ref, g3_ref, be3_ref, o_ref):
    ct = ct_ref[...]
    deg = jnp.sum(ct, axis=0, keepdims=True) + 2.0
    dis = jax.lax.rsqrt(deg)
    h = _agg_bn(hw1_ref[...], ct, deg, dis, g1_ref[...], be1_ref[...])
    h = _agg_bn(_transform(h, wt2_ref[...], F), ct, deg, dis,
                g2_ref[...], be2_ref[...])
    o_ref[...] = _agg_bn(_transform(h, wt3_ref[...], F), ct, deg, dis,
                         g3_ref[...], be3_ref[...])


def kernel(x, edge_index, W1, b1, g1, be1, W2, b2, g2, be2, W3, b3, g3, be3):
    eidx = edge_index[0] * N + edge_index[1]
    ct = _build_c(eidx).reshape(N, N)
    h0 = x.reshape(G * C0, N)
    hw1 = pl.pallas_call(
        _t1_body,
        out_shape=jax.ShapeDtypeStruct((G * F, N), jnp.float32),
    )(h0, W1.T)
    H = pl.pallas_call(
        _fused_body,
        out_shape=jax.ShapeDtypeStruct((G * F, N), jnp.float32),
        input_output_aliases={0: 0},
    )(hw1, ct, W2.T, W3.T, g1, be1, g2, be2, g3, be3)
    return H.reshape(G, F, N)


# speedup vs baseline: 1.6123x; 1.0355x over previous
import dataclasses
import functools

import jax
import jax.numpy as jnp
from jax import lax
from jax.experimental import pallas as pl
from jax.experimental.pallas import tpu as pltpu
from jax.experimental.pallas import tpu_sc as plsc

N = 2048
G = 32
C0 = 64
F = 32
E = 65536

NS = 16
NW = 2 * NS
ROWS_W = N // NW
HALF_ROWS = ROWS_W // 2
HALF_W = HALF_ROWS * N
EPIECE = 16384
NPIECE = E // EPIECE


def _build_c(eidx):
    cp = pltpu.CompilerParams()
    if "needs_layout_passes" in pltpu.CompilerParams.__dataclass_fields__:
        cp = dataclasses.replace(cp, needs_layout_passes=False)

    @functools.partial(
        pl.kernel,
        out_type=jax.ShapeDtypeStruct((N * N,), jnp.float32),
        mesh=plsc.VectorSubcoreMesh(core_axis_name="c", subcore_axis_name="s"),
        compiler_params=cp,
        scratch_types=[
            pltpu.VMEM((HALF_W,), jnp.float32),
            pltpu.VMEM((EPIECE,), jnp.int32),
            pltpu.VMEM((EPIECE,), jnp.int32),
            pltpu.VMEM_SHARED((E,), jnp.int32),
            pltpu.SemaphoreType.DMA,
            pltpu.SemaphoreType.DMA,
            pltpu.SemaphoreType.DMA,
        ],
    )
    def k(eidx_hbm, c_hbm, cbuf, eb0, eb1, eshared, sem0, sem1, semo):
        sid = lax.axis_index("s")
        wid = lax.axis_index("c") * NS + sid
        zero16 = jnp.zeros((16,), jnp.float32)
        one16 = jnp.ones((16,), jnp.float32)
        bufs = (eb0, eb1)
        sems = (sem0, sem1)

        @pl.when(sid == 0)
        def _():
            pltpu.async_copy(eidx_hbm, eshared, semo).wait()

        plsc.subcore_barrier()

        @pl.loop(0, 2)
        def _(half):
            basew = (wid * ROWS_W + half * HALF_ROWS) * N

            @pl.loop(0, HALF_W, step=64)
            def _(off):
                for j in range(0, 64, 16):
                    cbuf[pl.ds(off + j, 16)] = zero16

            cps = [None] * NPIECE
            cps[0] = pltpu.async_copy(
                eshared.at[pl.ds(0, EPIECE)], eb0, sem0)
            for i in range(NPIECE):
                if i + 1 < NPIECE:
                    cps[i + 1] = pltpu.async_copy(
                        eshared.at[pl.ds((i + 1) * EPIECE, EPIECE)],
                        bufs[(i + 1) % 2], sems[(i + 1) % 2])
                cps[i].wait()
                buf = bufs[i % 2]

                @pl.loop(0, EPIECE, step=64)
                def _(v, buf=buf):
                    for j in range(0, 64, 16):
                        r = buf[pl.ds(v + j, 16)] - basew
                        mask = (r >= 0) & (r < HALF_W)
                        plsc.addupdate_scatter(cbuf, [r], one16, mask=mask)

            pltpu.async_copy(
                cbuf, c_hbm.at[pl.ds(basew, HALF_W)], semo
            ).wait()

    return k(eidx)


def _transform(h, wt, ci):
    return jnp.concatenate(
        [jnp.dot(wt, h[g * ci:(g + 1) * ci, :],
                 preferred_element_type=jnp.float32) for g in range(G)],
        axis=0)


def _agg_bn(hw, ct, deg, dis, gam, bet):
    out = jnp.dot(dis * hw, ct, preferred_element_type=jnp.float32)
    out = dis * out + (2.0 / deg) * hw
    o3 = out.reshape(G, F, N)
    mu = jnp.mean(o3, axis=(0, 2))
    var = jnp.mean(o3 * o3, axis=(0, 2)) - mu * mu
    scale = gam * jax.lax.rsqrt(var + 1e-5)
    shift = bet - mu * scale
    o3 = jnp.maximum(o3 * scale[None, :, None] + shift[None, :, None], 0.0)
    return o3.reshape(G * F, N)


def _t1_body(x_ref, wt_ref, o_ref):
    o_ref[...] = _transform(x_ref[...], wt_ref[...], C0)


def _fused_body(hw1_ref, ct_ref, wt2_ref, wt3_ref, g1_ref, be1_ref,
                g2_ref, be2_ref, g3_ref, be3_ref, o_ref):
    ct = ct_ref[...]
    deg = jnp.sum(ct, axis=0, keepdims=True) + 2.0
    dis = jax.lax.rsqrt(deg)
    h = _agg_bn(hw1_ref[...], ct, deg, dis, g1_ref[...], be1_ref[...])
    h = _agg_bn(_transform(h, wt2_ref[...], F), ct, deg, dis,
                g2_ref[...], be2_ref[...])
    o_ref[...] = _agg_bn(_transform(h, wt3_ref[...], F), ct, deg, dis,
                         g3_ref[...], be3_ref[...])


def kernel(x, edge_index, W1, b1, g1, be1, W2, b2, g2, be2, W3, b3, g3, be3):
    eidx = edge_index[0] * N + edge_index[1]
    ct = _build_c(eidx).reshape(N, N)
    h0 = x.reshape(G * C0, N)
    hw1 = pl.pallas_call(
        _t1_body,
        out_shape=jax.ShapeDtypeStruct((G * F, N), jnp.float32),
    )(h0, W1.T)
    H = pl.pallas_call(
        _fused_body,
        out_shape=jax.ShapeDtypeStruct((G * F, N), jnp.float32),
        input_output_aliases={0: 0},
    )(hw1, ct, W2.T, W3.T, g1, be1, g2, be2, g3, be3)
    return H.reshape(G, F, N)


# speedup vs baseline: 1.6132x; 1.0005x over previous
import dataclasses
import functools

import jax
import jax.numpy as jnp
from jax import lax
from jax.experimental import pallas as pl
from jax.experimental.pallas import tpu as pltpu
from jax.experimental.pallas import tpu_sc as plsc

N = 2048
G = 32
C0 = 64
F = 32
E = 65536

NS = 16
NW = 2 * NS
ROWS_W = N // NW
HALF_ROWS = ROWS_W // 2
HALF_W = HALF_ROWS * N
EPIECE = 16384
NPIECE = E // EPIECE


def _build_c(eidx):
    cp = pltpu.CompilerParams()
    if "needs_layout_passes" in pltpu.CompilerParams.__dataclass_fields__:
        cp = dataclasses.replace(cp, needs_layout_passes=False)

    @functools.partial(
        pl.kernel,
        out_type=jax.ShapeDtypeStruct((N * N,), jnp.float32),
        mesh=plsc.VectorSubcoreMesh(core_axis_name="c", subcore_axis_name="s"),
        compiler_params=cp,
        scratch_types=[
            pltpu.VMEM((HALF_W,), jnp.float32),
            pltpu.VMEM((EPIECE,), jnp.int32),
            pltpu.VMEM((EPIECE,), jnp.int32),
            pltpu.VMEM_SHARED((E,), jnp.int32),
            pltpu.SemaphoreType.DMA,
            pltpu.SemaphoreType.DMA,
            pltpu.SemaphoreType.DMA,
        ],
    )
    def k(eidx_hbm, c_hbm, cbuf, eb0, eb1, eshared, sem0, sem1, semo):
        sid = lax.axis_index("s")
        wid = lax.axis_index("c") * NS + sid
        zero16 = jnp.zeros((16,), jnp.float32)
        one16 = jnp.ones((16,), jnp.float32)
        bufs = (eb0, eb1)
        sems = (sem0, sem1)

        @pl.when(sid == 0)
        def _():
            pltpu.async_copy(eidx_hbm, eshared, semo).wait()

        plsc.subcore_barrier()

        @pl.loop(0, 2)
        def _(half):
            basew = (wid * ROWS_W + half * HALF_ROWS) * N

            @pl.loop(0, HALF_W, step=64)
            def _(off):
                for j in range(0, 64, 16):
                    cbuf[pl.ds(off + j, 16)] = zero16

            cps = [None] * NPIECE
            cps[0] = pltpu.async_copy(
                eshared.at[pl.ds(0, EPIECE)], eb0, sem0)
            for i in range(NPIECE):
                if i + 1 < NPIECE:
                    cps[i + 1] = pltpu.async_copy(
                        eshared.at[pl.ds((i + 1) * EPIECE, EPIECE)],
                        bufs[(i + 1) % 2], sems[(i + 1) % 2])
                cps[i].wait()
                buf = bufs[i % 2]

                @pl.loop(0, EPIECE, step=64)
                def _(v, buf=buf):
                    for j in range(0, 64, 16):
                        r = buf[pl.ds(v + j, 16)] - basew
                        mask = r.astype(jnp.uint32) < jnp.uint32(HALF_W)
                        plsc.addupdate_scatter(cbuf, [r], one16, mask=mask)

            pltpu.async_copy(
                cbuf, c_hbm.at[pl.ds(basew, HALF_W)], semo
            ).wait()

    return k(eidx)


def _transform(h, wt, ci):
    return jnp.concatenate(
        [jnp.dot(wt, h[g * ci:(g + 1) * ci, :],
                 preferred_element_type=jnp.float32) for g in range(G)],
        axis=0)


def _agg_bn(hw, ct, deg, dis, gam, bet):
    out = jnp.dot(dis * hw, ct, preferred_element_type=jnp.float32)
    out = dis * out + (2.0 / deg) * hw
    o3 = out.reshape(G, F, N)
    mu = jnp.mean(o3, axis=(0, 2))
    var = jnp.mean(o3 * o3, axis=(0, 2)) - mu * mu
    scale = gam * jax.lax.rsqrt(var + 1e-5)
    shift = bet - mu * scale
    o3 = jnp.maximum(o3 * scale[None, :, None] + shift[None, :, None], 0.0)
    return o3.reshape(G * F, N)


def _t1_body(x_ref, wt_ref, o_ref):
    o_ref[...] = _transform(x_ref[...], wt_ref[...], C0)


def _fused_body(hw1_ref, ct_ref, wt2_ref, wt3_ref, g1_ref, be1_ref,
                g2_ref, be2_ref, g3_ref, be3_ref, o_ref):
    ct = ct_ref[...]
    deg = jnp.sum(ct, axis=0, keepdims=True) + 2.0
    dis = jax.lax.rsqrt(deg)
    h = _agg_bn(hw1_ref[...], ct, deg, dis, g1_ref[...], be1_ref[...])
    h = _agg_bn(_transform(h, wt2_ref[...], F), ct, deg, dis,
                g2_ref[...], be2_ref[...])
    o_ref[...] = _agg_bn(_transform(h, wt3_ref[...], F), ct, deg, dis,
                         g3_ref[...], be3_ref[...])


def kernel(x, edge_index, W1, b1, g1, be1, W2, b2, g2, be2, W3, b3, g3, be3):
    eidx = edge_index[0] * N + edge_index[1]
    ct = _build_c(eidx).reshape(N, N)
    h0 = x.reshape(G * C0, N)
    hw1 = pl.pallas_call(
        _t1_body,
        out_shape=jax.ShapeDtypeStruct((G * F, N), jnp.float32),
    )(h0, W1.T)
    H = pl.pallas_call(
        _fused_body,
        out_shape=jax.ShapeDtypeStruct((G * F, N), jnp.float32),
        input_output_aliases={0: 0},
    )(hw1, ct, W2.T, W3.T, g1, be1, g2, be2, g3, be3)
    return H.reshape(G, F, N)
